# Initial kernel scaffold; baseline (speedup 1.0000x reference)
import jax
import jax.numpy as jnp
from jax.experimental import pallas as pl


def kernel(x, W):
    raise NotImplementedError("write your pallas kernel here")


# speedup vs baseline: 1.6154x; 1.6154x over previous
import functools

import jax
import jax.numpy as jnp
from jax import lax
from jax.experimental import pallas as pl
from jax.experimental.pallas import tpu as pltpu
from jax.experimental.pallas import tpu_sc as plsc

_N = 100000
_NW = 32
_CHUNK = 3136
_LAST = _N - (_NW - 1) * _CHUNK
_EXTRA = _CHUNK - _LAST


def _body(x_hbm, w_hbm, out_hbm, x_v, w_v, out_v):
    nc = 2
    wid = lax.axis_index("s") * nc + lax.axis_index("c")
    base = wid * _CHUNK

    pltpu.sync_copy(w_hbm, w_v)

    pltpu.sync_copy(
        x_hbm.at[pl.ds(2 * base, 2 * _LAST)], x_v.at[pl.ds(0, 2 * _LAST)]
    )

    @pl.when(wid < _NW - 1)
    def _():
        pltpu.sync_copy(
            x_hbm.at[pl.ds(2 * base + 2 * _LAST, 2 * _EXTRA)],
            x_v.at[pl.ds(2 * _LAST, 2 * _EXTRA)],
        )

    iota = lax.iota(jnp.int32, 16)
    iota2 = 2 * iota
    iota4 = 4 * iota
    n_iters = jnp.where(wid == _NW - 1, _LAST // 16, _CHUNK // 16)

    def step(i, carry):
        r16 = i * 16
        a = 2 * r16 + iota2
        idx = plsc.load_gather(x_v, [a])
        x1 = plsc.load_gather(x_v, [a + 1])
        wbase = idx * 3
        e0 = plsc.load_gather(w_v, [wbase])
        e1 = plsc.load_gather(w_v, [wbase + 1])
        e2 = plsc.load_gather(w_v, [wbase + 2])
        o = 4 * r16 + iota4
        plsc.store_scatter(out_v, [o], e0)
        plsc.store_scatter(out_v, [o + 1], e1)
        plsc.store_scatter(out_v, [o + 2], e2)
        plsc.store_scatter(out_v, [o + 3], x1.astype(jnp.float32))
        return carry

    lax.fori_loop(0, n_iters, step, 0)

    pltpu.sync_copy(
        out_v.at[pl.ds(0, 4 * _LAST)], out_hbm.at[pl.ds(4 * base, 4 * _LAST)]
    )

    @pl.when(wid < _NW - 1)
    def _():
        pltpu.sync_copy(
            out_v.at[pl.ds(4 * _LAST, 4 * _EXTRA)],
            out_hbm.at[pl.ds(4 * base + 4 * _LAST, 4 * _EXTRA)],
        )


_sc_call = pl.kernel(
    _body,
    out_type=jax.ShapeDtypeStruct((4 * _N,), jnp.float32),
    mesh=plsc.VectorSubcoreMesh(core_axis_name="c", subcore_axis_name="s"),
    scratch_types=[
        pltpu.VMEM((2 * _CHUNK,), jnp.int32),
        pltpu.VMEM((16,), jnp.float32),
        pltpu.VMEM((4 * _CHUNK,), jnp.float32),
    ],
    compiler_params=pltpu.CompilerParams(needs_layout_passes=False),
)


def kernel(x, W):
    xf = x.reshape(-1).astype(jnp.int32)
    w16 = jnp.pad(W.reshape(-1), (0, 7))
    return _sc_call(xf, w16).reshape(_N, 4)


# speedup vs baseline: 7.3075x; 4.5237x over previous
import jax
import jax.numpy as jnp
from jax import lax
from jax.experimental import pallas as pl
from jax.experimental.pallas import tpu as pltpu
from jax.experimental.pallas import tpu_sc as plsc

_N = 100000
_NW = 32
_CHUNK = 3136
_LAST = _N - (_NW - 1) * _CHUNK
_EXTRA = _CHUNK - _LAST


def _copy_piecewise(src, dst, base, wid):
    pltpu.sync_copy(src.at[pl.ds(base, _LAST)], dst.at[pl.ds(0, _LAST)])

    @pl.when(wid < _NW - 1)
    def _():
        pltpu.sync_copy(
            src.at[pl.ds(base + _LAST, _EXTRA)], dst.at[pl.ds(_LAST, _EXTRA)]
        )


def _copy_out_piecewise(src, dst, base, wid):
    pltpu.sync_copy(src.at[pl.ds(0, _LAST)], dst.at[pl.ds(base, _LAST)])

    @pl.when(wid < _NW - 1)
    def _():
        pltpu.sync_copy(
            src.at[pl.ds(_LAST, _EXTRA)], dst.at[pl.ds(base + _LAST, _EXTRA)]
        )


def _body(x0_hbm, x1_hbm, w_hbm, o0_hbm, o1_hbm, o2_hbm, o3_hbm,
          x0_v, x1_v, w_v, o0_v, o1_v, o2_v, o3_v):
    nc = 2
    wid = lax.axis_index("s") * nc + lax.axis_index("c")
    base = wid * _CHUNK

    pltpu.sync_copy(w_hbm, w_v)
    _copy_piecewise(x0_hbm, x0_v, base, wid)
    _copy_piecewise(x1_hbm, x1_v, base, wid)

    n_iters = jnp.where(wid == _NW - 1, _LAST // 16, _CHUNK // 16)

    def step(i, carry):
        s = pl.ds(i * 16, 16)
        idx = x0_v[s]
        x1 = x1_v[s]
        wbase = idx * 3
        o0_v[s] = plsc.load_gather(w_v, [wbase])
        o1_v[s] = plsc.load_gather(w_v, [wbase + 1])
        o2_v[s] = plsc.load_gather(w_v, [wbase + 2])
        o3_v[s] = x1.astype(jnp.float32)
        return carry

    lax.fori_loop(0, n_iters, step, 0)

    _copy_out_piecewise(o0_v, o0_hbm, base, wid)
    _copy_out_piecewise(o1_v, o1_hbm, base, wid)
    _copy_out_piecewise(o2_v, o2_hbm, base, wid)
    _copy_out_piecewise(o3_v, o3_hbm, base, wid)


_plane = jax.ShapeDtypeStruct((_N,), jnp.float32)
_sc_call = pl.kernel(
    _body,
    out_type=(_plane, _plane, _plane, _plane),
    mesh=plsc.VectorSubcoreMesh(core_axis_name="c", subcore_axis_name="s"),
    scratch_types=[
        pltpu.VMEM((_CHUNK,), jnp.int32),
        pltpu.VMEM((_CHUNK,), jnp.int32),
        pltpu.VMEM((16,), jnp.float32),
        pltpu.VMEM((_CHUNK,), jnp.float32),
        pltpu.VMEM((_CHUNK,), jnp.float32),
        pltpu.VMEM((_CHUNK,), jnp.float32),
        pltpu.VMEM((_CHUNK,), jnp.float32),
    ],
    compiler_params=pltpu.CompilerParams(needs_layout_passes=False),
)


def kernel(x, W):
    x0 = x[:, 0]
    x1 = x[:, 1]
    w16 = jnp.pad(W.reshape(-1), (0, 7))
    o0, o1, o2, o3 = _sc_call(x0, x1, w16)
    return jnp.concatenate(
        (o0[:, None], o1[:, None], o2[:, None], o3[:, None]), axis=1
    )


# speedup vs baseline: 8.2327x; 1.1266x over previous
import jax
import jax.numpy as jnp
from jax import lax
from jax.experimental import pallas as pl
from jax.experimental.pallas import tpu as pltpu
from jax.experimental.pallas import tpu_sc as plsc

_N = 100000
_NW = 32
_CHUNK = 3136
_ITERS = _CHUNK // 16


def _body(x0_hbm, x1_hbm, w_hbm, o0_hbm, o1_hbm, o2_hbm, o3_hbm,
          x0_v, x1_v, w_v, o0_v, o1_v, o2_v, o3_v):
    nc = 2
    wid = lax.axis_index("s") * nc + lax.axis_index("c")
    base = jnp.where(wid == _NW - 1, _N - _CHUNK, wid * _CHUNK)

    pltpu.sync_copy(w_hbm, w_v)
    pltpu.sync_copy(x0_hbm.at[pl.ds(base, _CHUNK)], x0_v)
    pltpu.sync_copy(x1_hbm.at[pl.ds(base, _CHUNK)], x1_v)

    @plsc.parallel_loop(0, _ITERS, unroll=14)
    def _step(i):
        s = pl.ds(i * 16, 16)
        idx = x0_v[s]
        x1 = x1_v[s]
        wbase = idx * 3
        o0_v[s] = plsc.load_gather(w_v, [wbase])
        o1_v[s] = plsc.load_gather(w_v, [wbase + 1])
        o2_v[s] = plsc.load_gather(w_v, [wbase + 2])
        o3_v[s] = x1.astype(jnp.float32)

    pltpu.sync_copy(o0_v, o0_hbm.at[pl.ds(base, _CHUNK)])
    pltpu.sync_copy(o1_v, o1_hbm.at[pl.ds(base, _CHUNK)])
    pltpu.sync_copy(o2_v, o2_hbm.at[pl.ds(base, _CHUNK)])
    pltpu.sync_copy(o3_v, o3_hbm.at[pl.ds(base, _CHUNK)])


_plane = jax.ShapeDtypeStruct((_N,), jnp.float32)
_sc_call = pl.kernel(
    _body,
    out_type=(_plane, _plane, _plane, _plane),
    mesh=plsc.VectorSubcoreMesh(core_axis_name="c", subcore_axis_name="s"),
    scratch_types=[
        pltpu.VMEM((_CHUNK,), jnp.int32),
        pltpu.VMEM((_CHUNK,), jnp.int32),
        pltpu.VMEM((16,), jnp.float32),
        pltpu.VMEM((_CHUNK,), jnp.float32),
        pltpu.VMEM((_CHUNK,), jnp.float32),
        pltpu.VMEM((_CHUNK,), jnp.float32),
        pltpu.VMEM((_CHUNK,), jnp.float32),
    ],
    compiler_params=pltpu.CompilerParams(needs_layout_passes=False),
)


def kernel(x, W):
    x0 = x[:, 0]
    x1 = x[:, 1]
    w16 = jnp.pad(W.reshape(-1), (0, 7))
    o0, o1, o2, o3 = _sc_call(x0, x1, w16)
    return jnp.concatenate(
        (o0[:, None], o1[:, None], o2[:, None], o3[:, None]), axis=1
    )


# speedup vs baseline: 8.8228x; 1.0717x over previous
import jax
import jax.numpy as jnp
from jax import lax
from jax.experimental import pallas as pl
from jax.experimental.pallas import tpu as pltpu
from jax.experimental.pallas import tpu_sc as plsc

_N = 100000
_NW = 32
_CHUNK = 3136
_ITERS = _CHUNK // 16


def _body(x0_hbm, w_hbm, o0_hbm, o1_hbm, o2_hbm,
          x0_v, w_v, o0_v, o1_v, o2_v):
    nc = 2
    wid = lax.axis_index("s") * nc + lax.axis_index("c")
    base = jnp.where(wid == _NW - 1, _N - _CHUNK, wid * _CHUNK)

    pltpu.sync_copy(w_hbm, w_v)
    pltpu.sync_copy(x0_hbm.at[pl.ds(base, _CHUNK)], x0_v)

    @plsc.parallel_loop(0, _ITERS, unroll=14)
    def _step(i):
        s = pl.ds(i * 16, 16)
        wbase = x0_v[s] * 3
        o0_v[s] = plsc.load_gather(w_v, [wbase])
        o1_v[s] = plsc.load_gather(w_v, [wbase + 1])
        o2_v[s] = plsc.load_gather(w_v, [wbase + 2])

    pltpu.sync_copy(o0_v, o0_hbm.at[pl.ds(base, _CHUNK)])
    pltpu.sync_copy(o1_v, o1_hbm.at[pl.ds(base, _CHUNK)])
    pltpu.sync_copy(o2_v, o2_hbm.at[pl.ds(base, _CHUNK)])


_plane = jax.ShapeDtypeStruct((_N,), jnp.float32)
_sc_call = pl.kernel(
    _body,
    out_type=(_plane, _plane, _plane),
    mesh=plsc.VectorSubcoreMesh(core_axis_name="c", subcore_axis_name="s"),
    scratch_types=[
        pltpu.VMEM((_CHUNK,), jnp.int32),
        pltpu.VMEM((16,), jnp.float32),
        pltpu.VMEM((_CHUNK,), jnp.float32),
        pltpu.VMEM((_CHUNK,), jnp.float32),
        pltpu.VMEM((_CHUNK,), jnp.float32),
    ],
    compiler_params=pltpu.CompilerParams(needs_layout_passes=False),
)


def kernel(x, W):
    x0 = x[:, 0]
    w16 = jnp.pad(W.reshape(-1), (0, 7))
    o0, o1, o2 = _sc_call(x0, w16)
    o3 = x[:, 1].astype(jnp.float32)
    return jnp.concatenate(
        (o0[:, None], o1[:, None], o2[:, None], o3[:, None]), axis=1
    )
